# baseline (speedup 1.0000x reference)
import functools

import numpy as np
import jax
import jax.numpy as jnp
from jax import lax
from jax.experimental import pallas as pl
from jax.experimental.pallas import tpu as pltpu
from jax.experimental.pallas import tpu_sc as plsc

B, C, H, W, N = 16, 80, 128, 128, 32
NCAND = 9 * N
ONE_OFF = float(np.exp(np.float32(-0.5)))
TWO_OFF = float(np.exp(np.float32(-1.0)))
CORNERS = [(0, 0), (0, 2), (2, 0), (2, 2)]
EDGES = [(0, 1), (1, 0), (1, 2), (2, 1)]
ALL_OFFS = CORNERS + EDGES + [(1, 1)]
GBUF_WORDS = C * 32 * 32
FO_G, FO_OWN, FO_P, FO_ROW = 0, 384, 768, 1152
MO_OFF, MO_SZ, MO_OCC, MO_ROW = 0, 128, 256, 384


def _splat(x):
    return jnp.full((16,), x, jnp.float32)


def _sc_body(boxes_hbm, cls_hbm, clsp_hbm, offp_hbm, szp_hbm,
             focal_out, cent_out,
             box_v, clsv, gbuf, stage, mq, mst, sem):
    sid = lax.axis_index("s")
    iota = lax.iota(jnp.int32, 16)

    b = sid
    ic1 = pltpu.async_copy(boxes_hbm.at[b], box_v, sem)
    ic2 = pltpu.async_copy(cls_hbm.at[b], clsv, sem)
    ic1.wait()
    ic2.wait()
    base_l = []
    gid_l = []
    m_c = []
    m_n = []
    cell_l, offx_l, offy_l, szx_l, szy_l, oid_l = [], [], [], [], [], []
    for v in range(2):
        kp4 = (iota + v * 16) * 4
        x0 = plsc.load_gather(box_v, [kp4])
        y0 = plsc.load_gather(box_v, [kp4 + 1])
        x1 = plsc.load_gather(box_v, [kp4 + 2])
        y1 = plsc.load_gather(box_v, [kp4 + 3])
        cls_vec = clsv[pl.ds(v * 16, 16)]
        valid = cls_vec != -1
        ch = jnp.where(valid, cls_vec, 0)
        dx = x1 - x0
        dy = y1 - y0
        cx = dx >> 3
        cy = dy >> 3
        inb = valid & (cx - 1 >= 0) & (cy - 1 >= 0) & (cx + 1 < H) & (cy + 1 < W)
        base_l.append(ch * 1024 + cx * 32 + cy - 66)
        gid_l.append(b * (C * H * W) + ch * (H * W) + cx * W + cy - W - 1)
        m_c.append(valid)
        m_n.append(inb)
        validf = jnp.where(valid, 1.0, 0.0)
        cell_l.append(cx * 32 + cy)
        offx_l.append((dx & 7).astype(jnp.float32) * 0.125 * validf)
        offy_l.append((dy & 7).astype(jnp.float32) * 0.125 * validf)
        szx_l.append(dx.astype(jnp.float32) * validf)
        szy_l.append(dy.astype(jnp.float32) * validf)
        oid_l.append(b * (2 * H * W) + cx * W + cy)

    def _koff_geom(koff):
        j = koff - 4
        ox = jnp.where(koff < 4, (koff // 2) * 2,
                       jnp.where(koff < 8, (j + 1) // 2, 1))
        oy_e = 1 + jnp.asarray(j == 2, jnp.int32) - jnp.asarray(j == 1, jnp.int32)
        oy = jnp.where(koff < 4, (koff % 2) * 2, jnp.where(koff < 8, oy_e, 1))
        return ox * 32 + oy, ox * W + oy

    def _cand(koff, v):
        doff, goff = _koff_geom(koff)
        mask = jnp.where(koff == 8, m_c[v], m_n[v])
        return base_l[v] + doff, gid_l[v] + goff, mask

    cps = []
    for koff in range(9):
        for v in range(2):
            lidx, gidx, mask = _cand(koff, v)
            gidx = jnp.where(mask, gidx, 0)
            cps.append(pltpu.async_copy(
                clsp_hbm.at[gidx], stage.at[pl.ds(FO_P + (koff * 2 + v) * 16, 16)],
                sem))
    for comp in range(2):
        for v in range(2):
            oidx = oid_l[v] + comp * (H * W)
            cps.append(pltpu.async_copy(
                offp_hbm.at[oidx], mst.at[pl.ds(MO_OCC + 64 + comp * 32 + v * 16, 16)],
                sem))
            cps.append(pltpu.async_copy(
                szp_hbm.at[oidx], mst.at[pl.ds(MO_OCC + 128 + comp * 32 + v * 16, 16)],
                sem))

    def _id_scatter(koff, c):
        for v in range(2):
            lidx, _, mask = _cand(koff, v)
            sf = (iota + (koff * 32 + v * 16)).astype(jnp.float32)
            plsc.store_scatter(gbuf, [lidx], sf, mask=mask)
        return c

    def _id_read(koff, c):
        for v in range(2):
            lidx, _, mask = _cand(koff, v)
            sf = (iota + (koff * 32 + v * 16)).astype(jnp.float32)
            idr = plsc.load_gather(gbuf, [lidx], mask=mask)
            plsc.store_scatter(stage, [iota + (FO_OWN + koff * 32 + v * 16)],
                               jnp.where(mask & (idr == sf), 1.0, 0.0))
        return c

    def _val_scatter(koff, c):
        val = jnp.where(koff < 4, TWO_OFF, jnp.where(koff < 8, ONE_OFF, 1.0))
        for v in range(2):
            lidx, _, mask = _cand(koff, v)
            plsc.store_scatter(gbuf, [lidx],
                               jnp.broadcast_to(val, (16,)).astype(jnp.float32),
                               mask=mask)
        return c

    def _g_read(koff, c):
        for v in range(2):
            lidx, _, mask = _cand(koff, v)
            g = plsc.load_gather(gbuf, [lidx], mask=mask)
            plsc.store_scatter(stage, [iota + (FO_G + koff * 32 + v * 16)],
                               jnp.where(mask, g, 0.0))
        return c

    lax.fori_loop(0, 9, _id_scatter, 0, unroll=False)
    lax.fori_loop(0, 9, _id_read, 0, unroll=False)
    lax.fori_loop(0, 9, _val_scatter, 0, unroll=False)
    lax.fori_loop(0, 9, _g_read, 0, unroll=False)

    for v in range(2):
        plsc.store_scatter(mq, [cell_l[v]], _splat(0.0))
    for v in range(2):
        plsc.store_scatter(mq, [cell_l[v]], _splat(1.0), mask=m_c[v])
    occ = [plsc.load_gather(mq, [cell_l[v]]) for v in range(2)]
    for v in range(2):
        nf = (iota + v * 16).astype(jnp.float32)

        def _wid_store(lane, c, nf=nf, cell=cell_l[v]):
            plsc.store_scatter(mq, [cell], nf, mask=(iota == lane))
            return c

        lax.fori_loop(0, 16, _wid_store, 0, unroll=False)
    ownm = []
    for v in range(2):
        nf = (iota + v * 16).astype(jnp.float32)
        widr = plsc.load_gather(mq, [cell_l[v]])
        ownm.append(jnp.where(widr == nf, occ[v], 0.0))
    for cp in cps:
        cp.wait()
    for v in range(2):
        po0 = mst[pl.ds(MO_OCC + 64 + v * 16, 16)]
        po1 = mst[pl.ds(MO_OCC + 96 + v * 16, 16)]
        ps0 = mst[pl.ds(MO_OCC + 128 + v * 16, 16)]
        ps1 = mst[pl.ds(MO_OCC + 160 + v * 16, 16)]
        l1o = jnp.abs(po0 - offx_l[v]) + jnp.abs(po1 - offy_l[v])
        l1s = jnp.abs(ps0 - szx_l[v]) + jnp.abs(ps1 - szy_l[v])
        mst[pl.ds(MO_OFF + v * 16, 16)] = l1o * ownm[v]
        mst[pl.ds(MO_SZ + v * 16, 16)] = l1s * ownm[v]
        mst[pl.ds(MO_OCC + v * 16, 16)] = ownm[v]
    oc1 = pltpu.async_copy(stage, focal_out.at[b], sem)
    oc2 = pltpu.async_copy(mst.at[pl.ds(0, MO_ROW)], cent_out.at[b], sem)
    oc1.wait()
    oc2.wait()


_sc_call = functools.partial(
    pl.kernel,
    out_type=[
        jax.ShapeDtypeStruct((B, FO_ROW), jnp.float32),
        jax.ShapeDtypeStruct((B, MO_ROW), jnp.float32),
    ],
    mesh=plsc.VectorSubcoreMesh(core_axis_name="c", subcore_axis_name="s",
                                num_cores=1),
    compiler_params=pltpu.CompilerParams(needs_layout_passes=False),
    scratch_types=[
        pltpu.VMEM((4 * N,), jnp.int32),
        pltpu.VMEM((N,), jnp.int32),
        pltpu.VMEM((GBUF_WORDS,), jnp.float32),
        pltpu.VMEM((FO_ROW,), jnp.float32),
        pltpu.VMEM((32 * 32,), jnp.float32),
        pltpu.VMEM((MO_ROW + 192,), jnp.float32),
        pltpu.SemaphoreType.DMA,
    ],
)(_sc_body)


ROWS_PER_BLK = 32768
GRID = (B * C * H) // ROWS_PER_BLK


def _sum_body(x_ref, o_ref):
    @pl.when(pl.program_id(0) == 0)
    def _init():
        o_ref[...] = jnp.zeros((8, W), jnp.float32)

    p = jnp.clip(x_ref[...], 1e-4, 0.9999)
    t = p * p * jnp.log(1.0 - p)
    o_ref[...] += jnp.sum(t.reshape(ROWS_PER_BLK // 8, 8, W), axis=0)


def _final_body(part_ref, f_ref, m_ref, o_ref):
    base = jnp.sum(part_ref[...])
    fo = f_ref[...]
    g = fo[:, FO_G:FO_G + NCAND]
    own = fo[:, FO_OWN:FO_OWN + NCAND]
    pp = jnp.clip(fo[:, FO_P:FO_P + NCAND], 1e-4, 0.9999)
    basec = pp * pp * jnp.log(1.0 - pp)
    posc = (1.0 - pp) ** 4 * jnp.log(pp)
    act = jnp.where(g == 1.0, posc, (1.0 - g) ** 4 * basec)
    corr = jnp.sum(jnp.where(own > 0.5, act - basec, 0.0))
    focal = -(base + corr) / float(B * H * W)
    mo = m_ref[...]
    np2 = jnp.maximum(jnp.sum(mo[:, MO_OCC:MO_OCC + N]), 1.0)
    loss = focal + (jnp.sum(mo[:, MO_OFF:MO_OFF + N])
                    + 0.1 * jnp.sum(mo[:, MO_SZ:MO_SZ + N])) / np2
    o_ref[...] = jnp.full((1, W), loss, jnp.float32)


def kernel(cls_pred, offset_pred, size_pred, gt_box, gt_class):
    boxes = gt_box.reshape(B, 4 * N)
    clsp_flat = cls_pred.reshape(B * C * H * W)
    offp_flat = offset_pred.reshape(B * 2 * H * W)
    szp_flat = size_pred.reshape(B * 2 * H * W)

    focal_c, cent_c = _sc_call(boxes, gt_class, clsp_flat, offp_flat, szp_flat)

    partials = pl.pallas_call(
        _sum_body,
        grid=(GRID,),
        in_specs=[pl.BlockSpec((ROWS_PER_BLK, W), lambda i: (i, 0))],
        out_specs=pl.BlockSpec((8, W), lambda i: (0, 0)),
        out_shape=jax.ShapeDtypeStruct((8, W), jnp.float32),
    )(cls_pred.reshape(B * C * H, W))

    out = pl.pallas_call(
        _final_body,
        out_shape=jax.ShapeDtypeStruct((1, W), jnp.float32),
    )(partials, focal_c, cent_c)
    return out[0, 0]


# speedup vs baseline: 1.0082x; 1.0082x over previous
import functools

import numpy as np
import jax
import jax.numpy as jnp
from jax import lax
from jax.experimental import pallas as pl
from jax.experimental.pallas import tpu as pltpu
from jax.experimental.pallas import tpu_sc as plsc

B, C, H, W, N = 16, 80, 128, 128, 32
NCAND = 9 * N
ONE_OFF = float(np.exp(np.float32(-0.5)))
TWO_OFF = float(np.exp(np.float32(-1.0)))
CORNERS = [(0, 0), (0, 2), (2, 0), (2, 2)]
EDGES = [(0, 1), (1, 0), (1, 2), (2, 1)]
ALL_OFFS = CORNERS + EDGES + [(1, 1)]
GBUF_WORDS = C * 32 * 32
FO_G, FO_OWN, FO_P, FO_ROW = 0, 384, 768, 1152
MO_OFF, MO_SZ, MO_OCC, MO_ROW = 0, 128, 256, 384


def _splat(x):
    return jnp.full((16,), x, jnp.float32)


def _sc_body(boxes_hbm, cls_hbm, clsp_hbm, offp_hbm, szp_hbm,
             focal_out, cent_out,
             box_v, clsv, gbuf, stage, mq, mst, sem):
    sid = lax.axis_index("s")
    iota = lax.iota(jnp.int32, 16)

    b = sid
    ic1 = pltpu.async_copy(boxes_hbm.at[b], box_v, sem)
    ic2 = pltpu.async_copy(cls_hbm.at[b], clsv, sem)
    ic1.wait()
    ic2.wait()
    base_l = []
    gid_l = []
    m_c = []
    m_n = []
    cell_l, offx_l, offy_l, szx_l, szy_l, oid_l = [], [], [], [], [], []
    for v in range(2):
        kp4 = (iota + v * 16) * 4
        x0 = plsc.load_gather(box_v, [kp4])
        y0 = plsc.load_gather(box_v, [kp4 + 1])
        x1 = plsc.load_gather(box_v, [kp4 + 2])
        y1 = plsc.load_gather(box_v, [kp4 + 3])
        cls_vec = clsv[pl.ds(v * 16, 16)]
        valid = cls_vec != -1
        ch = jnp.where(valid, cls_vec, 0)
        dx = x1 - x0
        dy = y1 - y0
        cx = dx >> 3
        cy = dy >> 3
        inb = valid & (cx - 1 >= 0) & (cy - 1 >= 0) & (cx + 1 < H) & (cy + 1 < W)
        base_l.append(ch * 1024 + cx * 32 + cy - 66)
        gid_l.append(b * (C * H * W) + ch * (H * W) + cx * W + cy - W - 1)
        m_c.append(valid)
        m_n.append(inb)
        validf = jnp.where(valid, 1.0, 0.0)
        cell_l.append(cx * 32 + cy)
        offx_l.append((dx & 7).astype(jnp.float32) * 0.125 * validf)
        offy_l.append((dy & 7).astype(jnp.float32) * 0.125 * validf)
        szx_l.append(dx.astype(jnp.float32) * validf)
        szy_l.append(dy.astype(jnp.float32) * validf)
        oid_l.append(b * (2 * H * W) + cx * W + cy)

    def _koff_geom(koff):
        j = koff - 4
        ox = jnp.where(koff < 4, (koff // 2) * 2,
                       jnp.where(koff < 8, (j + 1) // 2, 1))
        oy_e = 1 + jnp.asarray(j == 2, jnp.int32) - jnp.asarray(j == 1, jnp.int32)
        oy = jnp.where(koff < 4, (koff % 2) * 2, jnp.where(koff < 8, oy_e, 1))
        return ox * 32 + oy, ox * W + oy

    def _cand(koff, v):
        doff, goff = _koff_geom(koff)
        mask = jnp.where(koff == 8, m_c[v], m_n[v])
        return base_l[v] + doff, gid_l[v] + goff, mask

    cps = []
    for koff in range(9):
        for v in range(2):
            lidx, gidx, mask = _cand(koff, v)
            gidx = jnp.where(mask, gidx, 0)
            cps.append(pltpu.async_copy(
                clsp_hbm.at[gidx], stage.at[pl.ds(FO_P + (koff * 2 + v) * 16, 16)],
                sem))
    for comp in range(2):
        for v in range(2):
            oidx = oid_l[v] + comp * (H * W)
            cps.append(pltpu.async_copy(
                offp_hbm.at[oidx], mst.at[pl.ds(MO_OCC + 64 + comp * 32 + v * 16, 16)],
                sem))
            cps.append(pltpu.async_copy(
                szp_hbm.at[oidx], mst.at[pl.ds(MO_OCC + 128 + comp * 32 + v * 16, 16)],
                sem))

    def _id_scatter(koff, c):
        for v in range(2):
            lidx, _, mask = _cand(koff, v)
            sf = (iota + (koff * 32 + v * 16)).astype(jnp.float32)
            plsc.store_scatter(gbuf, [lidx], sf, mask=mask)
        return c

    def _id_read(koff, c):
        for v in range(2):
            lidx, _, mask = _cand(koff, v)
            sf = (iota + (koff * 32 + v * 16)).astype(jnp.float32)
            idr = plsc.load_gather(gbuf, [lidx], mask=mask)
            plsc.store_scatter(stage, [iota + (FO_OWN + koff * 32 + v * 16)],
                               jnp.where(mask & (idr == sf), 1.0, 0.0))
        return c

    def _val_scatter(koff, c):
        val = jnp.where(koff < 4, TWO_OFF, jnp.where(koff < 8, ONE_OFF, 1.0))
        for v in range(2):
            lidx, _, mask = _cand(koff, v)
            plsc.store_scatter(gbuf, [lidx],
                               jnp.broadcast_to(val, (16,)).astype(jnp.float32),
                               mask=mask)
        return c

    def _g_read(koff, c):
        for v in range(2):
            lidx, _, mask = _cand(koff, v)
            g = plsc.load_gather(gbuf, [lidx], mask=mask)
            plsc.store_scatter(stage, [iota + (FO_G + koff * 32 + v * 16)],
                               jnp.where(mask, g, 0.0))
        return c

    lax.fori_loop(0, 9, _id_scatter, 0, unroll=False)
    lax.fori_loop(0, 9, _id_read, 0, unroll=False)
    lax.fori_loop(0, 9, _val_scatter, 0, unroll=False)
    lax.fori_loop(0, 9, _g_read, 0, unroll=False)

    for v in range(2):
        plsc.store_scatter(mq, [cell_l[v]], _splat(0.0))
    for v in range(2):
        plsc.store_scatter(mq, [cell_l[v]], _splat(1.0), mask=m_c[v])
    occ = [plsc.load_gather(mq, [cell_l[v]]) for v in range(2)]
    for v in range(2):
        nf = (iota + v * 16).astype(jnp.float32)

        def _wid_store(lane, c, nf=nf, cell=cell_l[v]):
            plsc.store_scatter(mq, [cell], nf, mask=(iota == lane))
            return c

        lax.fori_loop(0, 16, _wid_store, 0, unroll=False)
    ownm = []
    for v in range(2):
        nf = (iota + v * 16).astype(jnp.float32)
        widr = plsc.load_gather(mq, [cell_l[v]])
        ownm.append(jnp.where(widr == nf, occ[v], 0.0))
    for cp in cps:
        cp.wait()
    for v in range(2):
        po0 = mst[pl.ds(MO_OCC + 64 + v * 16, 16)]
        po1 = mst[pl.ds(MO_OCC + 96 + v * 16, 16)]
        ps0 = mst[pl.ds(MO_OCC + 128 + v * 16, 16)]
        ps1 = mst[pl.ds(MO_OCC + 160 + v * 16, 16)]
        l1o = jnp.abs(po0 - offx_l[v]) + jnp.abs(po1 - offy_l[v])
        l1s = jnp.abs(ps0 - szx_l[v]) + jnp.abs(ps1 - szy_l[v])
        mst[pl.ds(MO_OFF + v * 16, 16)] = l1o * ownm[v]
        mst[pl.ds(MO_SZ + v * 16, 16)] = l1s * ownm[v]
        mst[pl.ds(MO_OCC + v * 16, 16)] = ownm[v]
    oc1 = pltpu.async_copy(stage, focal_out.at[b], sem)
    oc2 = pltpu.async_copy(mst.at[pl.ds(0, MO_ROW)], cent_out.at[b], sem)
    oc1.wait()
    oc2.wait()


_sc_call = functools.partial(
    pl.kernel,
    out_type=[
        jax.ShapeDtypeStruct((B, FO_ROW), jnp.float32),
        jax.ShapeDtypeStruct((B, MO_ROW), jnp.float32),
    ],
    mesh=plsc.VectorSubcoreMesh(core_axis_name="c", subcore_axis_name="s",
                                num_cores=1),
    compiler_params=pltpu.CompilerParams(needs_layout_passes=False),
    scratch_types=[
        pltpu.VMEM((4 * N,), jnp.int32),
        pltpu.VMEM((N,), jnp.int32),
        pltpu.VMEM((GBUF_WORDS,), jnp.float32),
        pltpu.VMEM((FO_ROW,), jnp.float32),
        pltpu.VMEM((32 * 32,), jnp.float32),
        pltpu.VMEM((MO_ROW + 192,), jnp.float32),
        pltpu.SemaphoreType.DMA,
    ],
)(_sc_body)


ROWS_PER_BLK = 20480
GRID = (B * C * H) // ROWS_PER_BLK


def _sum_body(x_ref, o_ref):
    @pl.when(pl.program_id(0) == 0)
    def _init():
        o_ref[...] = jnp.zeros((8, W), jnp.float32)

    p = jnp.clip(x_ref[...], 1e-4, 0.9999)
    t = p * p * jnp.log(1.0 - p)
    o_ref[...] += jnp.sum(t.reshape(ROWS_PER_BLK // 8, 8, W), axis=0)


def _final_body(part_ref, f_ref, m_ref, o_ref):
    base = jnp.sum(part_ref[...])
    fo = f_ref[...]
    g = fo[:, FO_G:FO_G + NCAND]
    own = fo[:, FO_OWN:FO_OWN + NCAND]
    pp = jnp.clip(fo[:, FO_P:FO_P + NCAND], 1e-4, 0.9999)
    basec = pp * pp * jnp.log(1.0 - pp)
    posc = (1.0 - pp) ** 4 * jnp.log(pp)
    act = jnp.where(g == 1.0, posc, (1.0 - g) ** 4 * basec)
    corr = jnp.sum(jnp.where(own > 0.5, act - basec, 0.0))
    focal = -(base + corr) / float(B * H * W)
    mo = m_ref[...]
    np2 = jnp.maximum(jnp.sum(mo[:, MO_OCC:MO_OCC + N]), 1.0)
    loss = focal + (jnp.sum(mo[:, MO_OFF:MO_OFF + N])
                    + 0.1 * jnp.sum(mo[:, MO_SZ:MO_SZ + N])) / np2
    o_ref[...] = jnp.full((1, W), loss, jnp.float32)


def kernel(cls_pred, offset_pred, size_pred, gt_box, gt_class):
    boxes = gt_box.reshape(B, 4 * N)
    clsp_flat = cls_pred.reshape(B * C * H * W)
    offp_flat = offset_pred.reshape(B * 2 * H * W)
    szp_flat = size_pred.reshape(B * 2 * H * W)

    focal_c, cent_c = _sc_call(boxes, gt_class, clsp_flat, offp_flat, szp_flat)

    partials = pl.pallas_call(
        _sum_body,
        grid=(GRID,),
        in_specs=[pl.BlockSpec((ROWS_PER_BLK, W), lambda i: (i, 0))],
        out_specs=pl.BlockSpec((8, W), lambda i: (0, 0)),
        out_shape=jax.ShapeDtypeStruct((8, W), jnp.float32),
    )(cls_pred.reshape(B * C * H, W))

    out = pl.pallas_call(
        _final_body,
        out_shape=jax.ShapeDtypeStruct((1, W), jnp.float32),
    )(partials, focal_c, cent_c)
    return out[0, 0]
